# baseline (speedup 1.0000x reference)
import functools
import numpy as np
import jax
import jax.numpy as jnp
from jax import lax
from jax.experimental import pallas as pl
from jax.experimental.pallas import tpu as pltpu

_NUM_STEPS = 50
_BETA_START = 0.0001
_BETA_END = 0.5


def _alpha_bar_np():
    beta = np.linspace(_BETA_START ** 0.5, _BETA_END ** 0.5, _NUM_STEPS) ** 2
    return np.cumprod(1.0 - beta)


def _rhe(x):
    y = jnp.floor(x)
    frac = x - y
    y_odd = jnp.floor(y * 0.5) * 2.0 != y
    return y + jnp.where((frac > 0.5) | ((frac == 0.5) & y_odd), 1.0, 0.0)


def _search_step(vb, kf, st):
    lo, hi, clo, chi = st
    f_lo = lax.bitcast_convert_type(lo, jnp.float32)
    f_hi = lax.bitcast_convert_type(hi, jnp.float32)
    t = (clo - kf) / jnp.maximum(clo - chi, 1.0)
    mi = lax.bitcast_convert_type(f_lo + t * (f_hi - f_lo), jnp.int32)
    mb = lo + (hi - lo) // 2
    m1 = jnp.clip(jnp.minimum(mi, mb), lo + 1, hi - 1)
    m2 = jnp.clip(jnp.maximum(mi, mb), lo + 1, hi - 1)
    c1 = jnp.sum(jnp.where(vb >= m1, 1.0, 0.0))
    c2 = jnp.sum(jnp.where(vb >= m2, 1.0, 0.0))
    in_hi = c2 >= kf
    in_lo = c1 < kf
    lo_n = jnp.where(in_hi, m2, jnp.where(in_lo, lo, m1))
    clo_n = jnp.where(in_hi, c2, jnp.where(in_lo, clo, c1))
    hi_n = jnp.where(in_hi, hi, jnp.where(in_lo, m1, m2))
    chi_n = jnp.where(in_hi, chi, jnp.where(in_lo, c1, c2))
    live = (hi - lo) > 1
    return (jnp.where(live, lo_n, lo), jnp.where(live, hi_n, hi),
            jnp.where(live, clo_n, clo), jnp.where(live, chi_n, chi))


def _loss_kernel(sa_ref, sb_ref, ratio_ref, od_ref, om_ref, noise_ref,
                 rand_ref, w1_ref, out_ref, *, inv_b, tie_iters, spg):
    b = pl.program_id(0)
    w1 = w1_ref[...]
    K, L = w1.shape
    n_tot = float(K * L)

    om_ts, od_ts, vbs, kfs, kkfs = [], [], [], [], []
    for s in range(spg):
        om_t = jnp.transpose(om_ref[s])
        od_t = jnp.transpose(od_ref[s])
        rf = rand_ref[s] * om_t
        vb = lax.bitcast_convert_type(rf, jnp.int32)
        num_obs = jnp.sum(om_t)
        kkf = _rhe(num_obs * ratio_ref[0, 0, s])
        om_ts.append(om_t)
        od_ts.append(od_t)
        vbs.append(vb)
        kkfs.append(kkf)
        kfs.append(jnp.maximum(kkf, 1.0))

    def vcond(sts):
        alive = [(st[1] - st[0]) > 1 for st in sts]
        p = alive[0]
        for a in alive[1:]:
            p = p | a
        return p

    def vbody(sts):
        return tuple(_search_step(vbs[s], kfs[s], sts[s])
                     for s in range(spg))

    init = tuple((jnp.int32(0), jnp.int32(0x3F800000),
                  jnp.float32(n_tot), jnp.float32(0.0))
                 for _ in range(spg))
    final = lax.while_loop(vcond, vbody, init)

    loss_sum = jnp.float32(0.0)
    for s in range(spg):
        v, _, c_v, c_gt = final[s]
        vb, om_t, od_t, kkf, kf = vbs[s], om_ts[s], od_ts[s], kkfs[s], kfs[s]
        n_gt = c_gt
        m_eq = c_v - c_gt
        r = kf - n_gt

        def tie_search(vb=vb, v=v, r=r):
            idx = (lax.broadcasted_iota(jnp.int32, (K, L), 0) * L
                   + lax.broadcasted_iota(jnp.int32, (K, L), 1))

            def tbody(_, lohi):
                lo, hi = lohi
                mid = (lo + hi) // 2
                c = jnp.sum(jnp.where((vb == v) & (idx <= mid), 1.0, 0.0))
                take = c >= r
                return jnp.where(take, lo, mid + 1), jnp.where(take, mid, hi)

            return lax.fori_loop(0, tie_iters, tbody,
                                 (jnp.int32(0), jnp.int32(K * L - 1)))[0]

        need_tie = (v > 0) & (r < m_eq) & (kkf > 0)
        i_thr = lax.cond(
            need_tie, tie_search,
            lambda v=v, kkf=kkf: jnp.where((v > 0) & (kkf > 0),
                                           jnp.int32(K * L - 1),
                                           jnp.int32(-1)))

        v_eff = jnp.where(kkf > 0, v, jnp.int32(0x7F000000))

        sa = sa_ref[0, 0, s]
        sb = sb_ref[0, 0, s]
        noise = noise_ref[s]
        CH = 8
        col = lax.broadcasted_iota(jnp.int32, (CH, L), 1)
        row = lax.broadcasted_iota(jnp.int32, (CH, L), 0) * L
        acc_num = jnp.zeros((CH, L), jnp.float32)
        acc_cnt = jnp.zeros((CH, L), jnp.float32)
        for r0 in range(0, K, CH):
            vb_c = vb[r0:r0 + CH, :]
            idx_c = row + (r0 * L) + col
            in_topk = (vb_c > v_eff) | ((vb_c == v_eff) & (idx_c <= i_thr))
            tgt = om_t[r0:r0 + CH, :] * jnp.where(in_topk | (vb_c == 0),
                                                  1.0, 0.0)
            noise_c = noise[r0:r0 + CH, :]
            noisy = sa * od_t[r0:r0 + CH, :] + sb * noise_c
            resid = noise_c - noisy * w1[r0:r0 + CH, :]
            acc_num = acc_num + tgt * resid * resid
            acc_cnt = acc_cnt + tgt
        num = jnp.sum(acc_num)
        cnt = jnp.sum(acc_cnt)
        loss_sum = loss_sum + num / (cnt + 1e-6)

    @pl.when(b == 0)
    def _():
        out_ref[0, 0] = 0.0

    out_ref[0, 0] += loss_sum * inv_b


def kernel(observed_data, observed_mask, timepoints, gt_mask, t, noise,
           rand_vals, sample_ratios, w1, w2):
    B, L, K = observed_data.shape
    spg = 2 if B % 2 == 0 else 1
    G = B // spg
    ab = jnp.asarray(_alpha_bar_np(), jnp.float32)[t]
    sa = jnp.sqrt(ab).reshape(G, 1, spg)
    sb = jnp.sqrt(1.0 - ab).reshape(G, 1, spg)
    ratios = sample_ratios.reshape(G, 1, spg).astype(jnp.float32)
    tie_iters = max(1, int(np.ceil(np.log2(K * L))))

    smem = lambda: pl.BlockSpec((1, 1, spg), lambda b: (b, 0, 0),
                                memory_space=pltpu.SMEM)
    body = functools.partial(_loss_kernel, inv_b=float(1.0 / B),
                             tie_iters=tie_iters, spg=spg)
    out = pl.pallas_call(
        body,
        grid=(G,),
        in_specs=[
            smem(), smem(), smem(),
            pl.BlockSpec((spg, L, K), lambda b: (b, 0, 0)),
            pl.BlockSpec((spg, L, K), lambda b: (b, 0, 0)),
            pl.BlockSpec((spg, K, L), lambda b: (b, 0, 0)),
            pl.BlockSpec((spg, K, L), lambda b: (b, 0, 0)),
            pl.BlockSpec((K, L), lambda b: (0, 0)),
        ],
        out_specs=pl.BlockSpec((1, 1), lambda b: (0, 0),
                               memory_space=pltpu.SMEM),
        out_shape=jax.ShapeDtypeStruct((1, 1), jnp.float32),
    )(sa, sb, ratios, observed_data, observed_mask, noise, rand_vals, w1)
    return out[0, 0]


# speedup vs baseline: 1.1008x; 1.1008x over previous
import functools
import numpy as np
import jax
import jax.numpy as jnp
from jax import lax
from jax.experimental import pallas as pl
from jax.experimental.pallas import tpu as pltpu

_NUM_STEPS = 50
_BETA_START = 0.0001
_BETA_END = 0.5


def _alpha_bar_np():
    beta = np.linspace(_BETA_START ** 0.5, _BETA_END ** 0.5, _NUM_STEPS) ** 2
    return np.cumprod(1.0 - beta)


def _loss_kernel(sa_ref, sb_ref, ratio_ref, od_ref, om_ref, noise_ref,
                 rand_ref, w1_ref, out_ref, *, inv_b, tie_iters):
    b = pl.program_id(0)
    om_t = jnp.transpose(om_ref[0])
    od_t = jnp.transpose(od_ref[0])
    rand = rand_ref[0]
    noise = noise_ref[0]
    w1 = w1_ref[...]
    K, L = rand.shape
    n_tot = float(K * L)

    rf = rand * om_t
    vb = lax.bitcast_convert_type(rf, jnp.int32)
    num_obs = jnp.sum(om_ref[0])
    ratio = ratio_ref[0, 0, 0]
    x = num_obs * ratio
    y = jnp.floor(x)
    frac = x - y
    y_odd = jnp.floor(y * 0.5) * 2.0 != y
    kkf = y + jnp.where((frac > 0.5) | ((frac == 0.5) & y_odd), 1.0, 0.0)
    kf = jnp.maximum(kkf, 1.0)

    def vcond(st):
        return (st[1] - st[0]) > 1

    def vbody(st):
        lo, hi, clo, chi = st
        f_lo = lax.bitcast_convert_type(lo, jnp.float32)
        f_hi = lax.bitcast_convert_type(hi, jnp.float32)
        t = (clo - kf) / jnp.maximum(clo - chi, 1.0)
        mi = lax.bitcast_convert_type(f_lo + t * (f_hi - f_lo), jnp.int32)
        mb = lo + (hi - lo) // 2
        m1 = jnp.clip(jnp.minimum(mi, mb), lo + 1, hi - 1)
        m2 = jnp.clip(jnp.maximum(mi, mb), lo + 1, hi - 1)
        c1 = jnp.sum(jnp.where(vb >= m1, 1.0, 0.0))
        c2 = jnp.sum(jnp.where(vb >= m2, 1.0, 0.0))
        in_hi = c2 >= kf
        in_lo = c1 < kf
        lo_n = jnp.where(in_hi, m2, jnp.where(in_lo, lo, m1))
        clo_n = jnp.where(in_hi, c2, jnp.where(in_lo, clo, c1))
        hi_n = jnp.where(in_hi, hi, jnp.where(in_lo, m1, m2))
        chi_n = jnp.where(in_hi, chi, jnp.where(in_lo, c1, c2))
        return (lo_n, hi_n, clo_n, chi_n)

    v, _, c_v, c_gt = lax.while_loop(
        vcond, vbody,
        (jnp.int32(0), jnp.int32(0x3F800000),
         jnp.float32(n_tot), jnp.float32(0.0)))
    n_gt = c_gt
    m_eq = c_v - c_gt
    r = kf - n_gt

    def tie_search():
        idx = (lax.broadcasted_iota(jnp.int32, (K, L), 0) * L
               + lax.broadcasted_iota(jnp.int32, (K, L), 1))

        def tbody(_, lohi):
            lo, hi = lohi
            mid = (lo + hi) // 2
            c = jnp.sum(jnp.where((vb == v) & (idx <= mid), 1.0, 0.0))
            take = c >= r
            return jnp.where(take, lo, mid + 1), jnp.where(take, mid, hi)

        return lax.fori_loop(0, tie_iters, tbody,
                             (jnp.int32(0), jnp.int32(K * L - 1)))[0]

    need_tie = (v > 0) & (r < m_eq) & (kkf > 0)
    i_thr = lax.cond(
        need_tie, tie_search,
        lambda: jnp.where((v > 0) & (kkf > 0),
                          jnp.int32(K * L - 1), jnp.int32(-1)))

    v_eff = jnp.where(kkf > 0, v, jnp.int32(0x7F000000))

    sa = sa_ref[0, 0, 0]
    sb = sb_ref[0, 0, 0]
    CH = 8
    col = lax.broadcasted_iota(jnp.int32, (CH, L), 1)
    row = lax.broadcasted_iota(jnp.int32, (CH, L), 0) * L
    acc_num = jnp.zeros((CH, L), jnp.float32)
    acc_cnt = jnp.zeros((CH, L), jnp.float32)
    for r0 in range(0, K, CH):
        vb_c = vb[r0:r0 + CH, :]
        idx_c = row + (r0 * L) + col
        in_topk = (vb_c > v_eff) | ((vb_c == v_eff) & (idx_c <= i_thr))
        tgt = om_t[r0:r0 + CH, :] * jnp.where(in_topk | (vb_c == 0), 1.0, 0.0)
        noise_c = noise[r0:r0 + CH, :]
        noisy = sa * od_t[r0:r0 + CH, :] + sb * noise_c
        resid = noise_c - noisy * w1[r0:r0 + CH, :]
        acc_num = acc_num + tgt * resid * resid
        acc_cnt = acc_cnt + tgt
    num = jnp.sum(acc_num)
    cnt = jnp.sum(acc_cnt)
    loss_b = num / (cnt + 1e-6)

    @pl.when(b == 0)
    def _():
        out_ref[0, 0] = 0.0

    out_ref[0, 0] += loss_b * inv_b


def kernel(observed_data, observed_mask, timepoints, gt_mask, t, noise,
           rand_vals, sample_ratios, w1, w2):
    B, L, K = observed_data.shape
    ab = jnp.asarray(_alpha_bar_np(), jnp.float32)[t]
    sa = jnp.sqrt(ab).reshape(B, 1, 1)
    sb = jnp.sqrt(1.0 - ab).reshape(B, 1, 1)
    ratios = sample_ratios.reshape(B, 1, 1).astype(jnp.float32)
    tie_iters = max(1, int(np.ceil(np.log2(K * L))))

    smem = lambda: pl.BlockSpec((1, 1, 1), lambda b: (b, 0, 0),
                                memory_space=pltpu.SMEM)
    body = functools.partial(_loss_kernel, inv_b=float(1.0 / B),
                             tie_iters=tie_iters)
    out = pl.pallas_call(
        body,
        grid=(B,),
        in_specs=[
            smem(), smem(), smem(),
            pl.BlockSpec((1, L, K), lambda b: (b, 0, 0)),
            pl.BlockSpec((1, L, K), lambda b: (b, 0, 0)),
            pl.BlockSpec((1, K, L), lambda b: (b, 0, 0)),
            pl.BlockSpec((1, K, L), lambda b: (b, 0, 0)),
            pl.BlockSpec((K, L), lambda b: (0, 0)),
        ],
        out_specs=pl.BlockSpec((1, 1), lambda b: (0, 0),
                               memory_space=pltpu.SMEM),
        out_shape=jax.ShapeDtypeStruct((1, 1), jnp.float32),
    )(sa, sb, ratios, observed_data, observed_mask, noise, rand_vals, w1)
    return out[0, 0]
